# baseline (speedup 1.0000x reference)
import functools

import jax
import jax.numpy as jnp
import numpy as np
from jax import lax
from jax.experimental import pallas as pl
from jax.experimental.pallas import tpu as pltpu
from jax.experimental.pallas import tpu_sc as plsc

DIM = 32
SMOOTHING = np.float32(0.01)
CT = 8


def _gdiff_body(nc, t_per_w, n_chunks, ncols,
                idx_hbm, w_hbm, e_hbm, out_hbm,
                idx_all, w0, w1, e0, e1, o0, o1,
                sem_w0, sem_w1, sem_e0, sem_e1):
    wid = lax.axis_index("s") * nc + lax.axis_index("c")
    t_base = pl.multiple_of(wid * t_per_w, t_per_w)
    row_ct = CT * ncols

    w_bufs = (w0, w1)
    e_bufs = (e0, e1)
    o_bufs = (o0, o1)
    sems_w = (sem_w0, sem_w1)
    sems_e = (sem_e0, sem_e1)

    pltpu.sync_copy(idx_hbm.at[pl.ds(t_base, t_per_w)], idx_all)

    def fire(c, k):
        for t in range(CT):
            row = c * CT + t
            dst = pl.ds(t * ncols, ncols)
            pltpu.async_copy(w_hbm.at[idx_all.at[row]], w_bufs[k].at[dst],
                             sems_w[k])
            pltpu.async_copy(e_hbm.at[idx_all.at[row]], e_bufs[k].at[dst],
                             sems_e[k])

    def drain(k):
        pltpu.make_async_copy(w_hbm.at[pl.ds(0, row_ct)], w_bufs[k],
                              sems_w[k]).wait()
        pltpu.make_async_copy(e_hbm.at[pl.ds(0, row_ct)], e_bufs[k],
                              sems_e[k]).wait()

    half = jnp.float32(0.5)
    three_half = jnp.float32(1.5)
    sign_mask = jnp.int32(-2147483648)

    def compute(k):
        wv, ev, ov = w_bufs[k], e_bufs[k], o_bufs[k]

        @plsc.parallel_loop(0, ncols, unroll=4)
        def col_body(j):
            for t in range(CT):
                row = t * ncols + j
                for h in (0, 16):
                    w = wv[row, pl.ds(h, 16)]
                    e = ev[row, pl.ds(h, 16)]
                    e2 = e + SMOOTHING * (w - e)
                    p = w * e2
                    a = jnp.abs(p)
                    bi = lax.bitcast_convert_type(a, jnp.int32)
                    bi = 0x5F3759DF - lax.shift_right_logical(bi, 1)
                    r = lax.bitcast_convert_type(bi, jnp.float32)
                    r = r * (three_half - half * a * r * r)
                    mag = a * r
                    sb = lax.bitcast_convert_type(w + e2, jnp.int32) & sign_mask
                    ob = lax.bitcast_convert_type(mag, jnp.int32) ^ sb
                    ov[pl.ds(row * DIM + h, 16)] = lax.bitcast_convert_type(
                        ob, jnp.float32)

    fire(0, 0)

    def pair_body(c0, carry):
        for k in (0, 1):
            c = c0 * 2 + k

            @pl.when(c + 1 < n_chunks)
            def _():
                fire(c + 1, 1 - k)

            drain(k)
            compute(k)
            ob = pl.multiple_of((t_base + c * CT) * ncols * DIM,
                                row_ct * DIM)
            pltpu.sync_copy(o_bufs[k], out_hbm.at[pl.ds(ob, row_ct * DIM)])
        return carry

    lax.fori_loop(0, n_chunks // 2, pair_body, 0)


def kernel(input, weight, weight_ema):
    nrows, ncols = input.shape
    idx2d = input.astype(jnp.int32)
    info = plsc.get_sparse_core_info()
    nc, ns = info.num_cores, info.num_subcores
    nw = nc * ns
    t_per_w = nrows // nw
    n_chunks = t_per_w // CT
    assert nrows % (nw * 2 * CT) == 0, (nrows, nw, CT)

    mesh = plsc.VectorSubcoreMesh(core_axis_name="c", subcore_axis_name="s")
    run = pl.kernel(
        functools.partial(_gdiff_body, nc, t_per_w, n_chunks, ncols),
        mesh=mesh,
        compiler_params=pltpu.CompilerParams(use_tc_tiling_on_sc=False),
        out_type=jax.ShapeDtypeStruct((nrows * ncols * DIM,), jnp.float32),
        scratch_types=[
            pltpu.VMEM((t_per_w, ncols), jnp.int32),
            pltpu.VMEM((CT * ncols, DIM), jnp.float32),
            pltpu.VMEM((CT * ncols, DIM), jnp.float32),
            pltpu.VMEM((CT * ncols, DIM), jnp.float32),
            pltpu.VMEM((CT * ncols, DIM), jnp.float32),
            pltpu.VMEM((CT * ncols * DIM,), jnp.float32),
            pltpu.VMEM((CT * ncols * DIM,), jnp.float32),
            pltpu.SemaphoreType.DMA,
            pltpu.SemaphoreType.DMA,
            pltpu.SemaphoreType.DMA,
            pltpu.SemaphoreType.DMA,
        ],
    )
    flat = run(idx2d, weight, weight_ema)
    return flat.reshape(nrows, ncols, DIM)


# speedup vs baseline: 1.1644x; 1.1644x over previous
import functools

import jax
import jax.numpy as jnp
import numpy as np
from jax import lax
from jax.experimental import pallas as pl
from jax.experimental.pallas import tpu as pltpu
from jax.experimental.pallas import tpu_sc as plsc

DIM = 32
SMOOTHING = np.float32(0.01)
TB = 128


def _gdiff_body(nc, t_per_w, ncols,
                idx_hbm, w_hbm, e_hbm, out_hbm,
                idx_all, iv0, iv1, w0, w1, e0, e1, o0, o1,
                sem_w0, sem_w1, sem_e0, sem_e1):
    wid = lax.axis_index("s") * nc + lax.axis_index("c")
    t_base = pl.multiple_of(wid * t_per_w, t_per_w)
    n_tb = t_per_w // TB
    n_chunks = n_tb * ncols

    iv_bufs = (iv0, iv1)
    w_bufs = (w0, w1)
    e_bufs = (e0, e1)
    o_bufs = (o0, o1)
    sems_w = (sem_w0, sem_w1)
    sems_e = (sem_e0, sem_e1)

    pltpu.sync_copy(idx_hbm.at[pl.ds(t_base, t_per_w)], idx_all)

    lanes = lax.iota(jnp.int32, 16)
    zeros = lanes & jnp.int32(0)
    a0 = lax.shift_right_logical(lanes, 3)
    a1 = a0 + jnp.int32(2)
    r_vec = lanes & jnp.int32(7)

    def fire(c, k):
        tbg = c // ncols
        j = c - tbg * ncols
        t0 = tbg * TB
        jv = zeros + j
        for g in range(TB // 16):
            rv = t0 + g * 16 + lanes
            vals = plsc.load_gather(idx_all, [rv, jv])
            iv_bufs[k][pl.ds(g * 16, 16)] = vals
        pltpu.async_copy(w_hbm.at[iv_bufs[k]], w_bufs[k], sems_w[k])
        pltpu.async_copy(e_hbm.at[iv_bufs[k]], e_bufs[k], sems_e[k])

    def drain(k):
        pltpu.make_async_copy(w_hbm.at[pl.ds(0, TB)], w_bufs[k],
                              sems_w[k]).wait()
        pltpu.make_async_copy(e_hbm.at[pl.ds(0, TB)], e_bufs[k],
                              sems_e[k]).wait()

    half = jnp.float32(0.5)
    three_half = jnp.float32(1.5)
    sign_mask = jnp.int32(-2147483648)

    def compute(k):
        wv, ev, ov = w_bufs[k], e_bufs[k], o_bufs[k]

        @plsc.parallel_loop(0, TB, unroll=2)
        def row_body(tt):
            cv = zeros + tt
            for h, av in ((0, a0), (16, a1)):
                w = wv[tt, pl.ds(h, 16)]
                e = ev[tt, pl.ds(h, 16)]
                e2 = e + SMOOTHING * (w - e)
                p = w * e2
                a = jnp.abs(p)
                bi = lax.bitcast_convert_type(a, jnp.int32)
                bi = 0x5F3759DF - lax.shift_right_logical(bi, 1)
                r = lax.bitcast_convert_type(bi, jnp.float32)
                r = r * (three_half - half * a * r * r)
                mag = a * r
                sb = lax.bitcast_convert_type(w + e2, jnp.int32) & sign_mask
                ob = lax.bitcast_convert_type(mag, jnp.int32) ^ sb
                plsc.store_scatter(ov, [av, r_vec, cv],
                                   lax.bitcast_convert_type(ob, jnp.float32))

    fire(0, 0)

    def pair_body(cp, carry):
        for k in (0, 1):
            c = cp * 2 + k

            @pl.when(c + 1 < n_chunks)
            def _():
                fire(c + 1, 1 - k)

            drain(k)
            compute(k)
            tbg = c // ncols
            j = c - tbg * ncols
            pltpu.sync_copy(
                o_bufs[k],
                out_hbm.at[j, :, wid * n_tb + tbg])
        return carry

    lax.fori_loop(0, n_chunks // 2, pair_body, 0)


def kernel(input, weight, weight_ema):
    nrows, ncols = input.shape
    idx2d = input.astype(jnp.int32)
    info = plsc.get_sparse_core_info()
    nc, ns = info.num_cores, info.num_subcores
    nw = nc * ns
    t_per_w = nrows // nw
    assert t_per_w % TB == 0 and ((t_per_w // TB) * ncols) % 2 == 0

    mesh = plsc.VectorSubcoreMesh(core_axis_name="c", subcore_axis_name="s")
    run = pl.kernel(
        functools.partial(_gdiff_body, nc, t_per_w, ncols),
        mesh=mesh,
        compiler_params=pltpu.CompilerParams(use_tc_tiling_on_sc=False,
                                             needs_layout_passes=False),
        out_type=jax.ShapeDtypeStruct(
            (ncols, DIM // 8, nrows // 128, 8, 128), jnp.float32),
        scratch_types=[
            pltpu.VMEM((t_per_w, ncols), jnp.int32),
            pltpu.VMEM((TB,), jnp.int32),
            pltpu.VMEM((TB,), jnp.int32),
            pltpu.VMEM((TB, DIM), jnp.float32),
            pltpu.VMEM((TB, DIM), jnp.float32),
            pltpu.VMEM((TB, DIM), jnp.float32),
            pltpu.VMEM((TB, DIM), jnp.float32),
            pltpu.VMEM((DIM // 8, 8, TB), jnp.float32),
            pltpu.VMEM((DIM // 8, 8, TB), jnp.float32),
            pltpu.SemaphoreType.DMA,
            pltpu.SemaphoreType.DMA,
            pltpu.SemaphoreType.DMA,
            pltpu.SemaphoreType.DMA,
        ],
    )
    out5 = run(idx2d, weight, weight_ema)
    return out5.transpose(2, 4, 0, 1, 3).reshape(nrows, ncols, DIM)
